# Initial kernel scaffold; baseline (speedup 1.0000x reference)
import jax
import jax.numpy as jnp
from jax.experimental import pallas as pl


def kernel(eb_input, eb_offset, W0, W1, W2, W3):
    raise NotImplementedError("write your pallas kernel here")


# speedup vs baseline: 72.2040x; 72.2040x over previous
import functools

import jax
import jax.numpy as jnp
from jax import lax
from jax.experimental import pallas as pl
from jax.experimental.pallas import tpu as pltpu
from jax.experimental.pallas import tpu_sc as plsc

D = 16
NC = 2
NS = 16
NW = NC * NS
GSZ = 128
CB = 4
C = GSZ * CB


@functools.lru_cache(maxsize=None)
def _make_kernel(n_idx):
    per_w = n_idx // NW
    rows_per_w = per_w // GSZ
    nch = per_w // C

    mesh = plsc.VectorSubcoreMesh(core_axis_name="c", subcore_axis_name="s")

    @functools.partial(
        pl.kernel,
        out_type=jax.ShapeDtypeStruct((NC, 2, D), jnp.float32),
        mesh=mesh,
        compiler_params=pltpu.CompilerParams(use_tc_tiling_on_sc=False),
        scratch_types=[
            pltpu.VMEM((CB, GSZ), jnp.int32),
            pltpu.VMEM((C, D), jnp.float32),
            pltpu.VMEM((C, D), jnp.float32),
            pltpu.VMEM((C, D), jnp.float32),
            pltpu.VMEM((C, D), jnp.float32),
            pltpu.VMEM((2, D), jnp.float32),
            pltpu.VMEM((NS, 2, D), jnp.float32),
            pltpu.VMEM_SHARED((NS, 2, D), jnp.float32),
            pltpu.SemaphoreType.DMA,
        ],
    )
    def k(idx_hbm, w0, w1, w2, w3, out_hbm,
          idx_v, r0, r1, r2, r3, part_v, red_v, shared, sem):
        cid = lax.axis_index("c")
        sid = lax.axis_index("s")
        wid = sid * NC + cid
        zero = jnp.zeros((D,), jnp.float32)

        def chunk_body(kc, carry):
            acc_a, acc_b = carry
            row0 = wid * rows_per_w + kc * CB
            pltpu.sync_copy(idx_hbm.at[pl.ds(row0, CB)], idx_v)
            cps = []
            for g in range(CB):
                ig = idx_v.at[g]
                sl = pl.ds(g * GSZ, GSZ)
                cps.append(pltpu.async_copy(w0.at[ig], r0.at[sl], sem))
                cps.append(pltpu.async_copy(w2.at[ig], r2.at[sl], sem))
                cps.append(pltpu.async_copy(w1.at[ig], r1.at[sl], sem))
                cps.append(pltpu.async_copy(w3.at[ig], r3.at[sl], sem))
            for cp in cps:
                cp.wait()

            unroll = 8

            def row_body(t, carry2):
                a, b = carry2
                base = t * unroll
                for u in range(unroll):
                    j = base + u
                    a = a + r0[j] + r2[j]
                    b = b + r1[j] + r3[j]
                return a, b

            return lax.fori_loop(0, C // unroll, row_body, (acc_a, acc_b))

        acc_a, acc_b = lax.fori_loop(0, nch, chunk_body, (zero, zero))

        part_v[0] = acc_a
        part_v[1] = acc_b
        pltpu.sync_copy(part_v, shared.at[sid])
        plsc.subcore_barrier()

        @pl.when(sid == 0)
        def _():
            pltpu.sync_copy(shared, red_v)
            a = jnp.zeros((D,), jnp.float32)
            b = jnp.zeros((D,), jnp.float32)
            for i in range(NS):
                a = a + red_v[i, 0]
                b = b + red_v[i, 1]
            part_v[0] = a
            part_v[1] = b
            pltpu.sync_copy(part_v, out_hbm.at[cid])

    return k


def kernel(eb_input, eb_offset, W0, W1, W2, W3):
    del eb_offset
    n = eb_input.shape[0]
    idx2d = eb_input.astype(jnp.int32).reshape(n // GSZ, GSZ)
    parts = _make_kernel(n)(idx2d, W0, W1, W2, W3)
    return parts.sum(axis=0).reshape(2 * D)


# speedup vs baseline: 76.9650x; 1.0659x over previous
import functools

import jax
import jax.numpy as jnp
from jax import lax
from jax.experimental import pallas as pl
from jax.experimental.pallas import tpu as pltpu
from jax.experimental.pallas import tpu_sc as plsc

D = 16
NC = 2
NS = 16
NW = NC * NS
GSZ = 128
CB = 4
C = GSZ * CB
U = 8
NACC = 4


@functools.lru_cache(maxsize=None)
def _make_kernel(n_idx):
    per_w = n_idx // NW
    rows_per_w = per_w // GSZ
    nch = per_w // C
    assert nch % 2 == 0

    mesh = plsc.VectorSubcoreMesh(core_axis_name="c", subcore_axis_name="s")

    rows_t = pltpu.VMEM((C, D), jnp.float32)

    @functools.partial(
        pl.kernel,
        out_type=jax.ShapeDtypeStruct((NC, 2, D), jnp.float32),
        mesh=mesh,
        compiler_params=pltpu.CompilerParams(use_tc_tiling_on_sc=False),
        scratch_types=[
            pltpu.VMEM((rows_per_w, GSZ), jnp.int32),
            rows_t, rows_t, rows_t, rows_t,
            rows_t, rows_t, rows_t, rows_t,
            pltpu.VMEM((2, D), jnp.float32),
            pltpu.VMEM((NS, 2, D), jnp.float32),
            pltpu.VMEM_SHARED((NS, 2, D), jnp.float32),
            pltpu.SemaphoreType.DMA,
            pltpu.SemaphoreType.DMA,
        ],
    )
    def k(idx_hbm, w0, w1, w2, w3, out_hbm,
          idx_all, r0a, r1a, r2a, r3a, r0b, r1b, r2b, r3b,
          part_v, red_v, shared, semg0, semg1):
        cid = lax.axis_index("c")
        sid = lax.axis_index("s")
        wid = sid * NC + cid
        zero = jnp.zeros((D,), jnp.float32)

        ws = (w0, w1, w2, w3)
        bufs = ((r0a, r1a, r2a, r3a), (r0b, r1b, r2b, r3b))
        sems = (semg0, semg1)

        pltpu.sync_copy(idx_hbm.at[pl.ds(wid * rows_per_w, rows_per_w)],
                        idx_all)

        def fire(kc, par):
            rb, sem = bufs[par], sems[par]
            for g in range(CB):
                ig = idx_all.at[kc * CB + g]
                sl = pl.ds(g * GSZ, GSZ)
                for t in range(4):
                    pltpu.async_copy(ws[t].at[ig], rb[t].at[sl], sem)

        def wait(kc, par):
            rb, sem = bufs[par], sems[par]
            for g in range(CB):
                ig = idx_all.at[kc * CB + g]
                sl = pl.ds(g * GSZ, GSZ)
                for t in range(4):
                    pltpu.make_async_copy(ws[t].at[ig], rb[t].at[sl],
                                          sem).wait()

        def accum(par, carry):
            rb = bufs[par]

            def row_body(tt, carry2):
                aa, bb = list(carry2[0]), list(carry2[1])
                base = tt * U
                for u in range(U):
                    j = base + u
                    q = u % NACC
                    aa[q] = aa[q] + (rb[0][j] + rb[2][j])
                    bb[q] = bb[q] + (rb[1][j] + rb[3][j])
                return (tuple(aa), tuple(bb))

            return lax.fori_loop(0, C // U, row_body, carry)

        def half(kc, par, carry):
            @pl.when(kc + 1 < nch)
            def _():
                fire(kc + 1, 1 - par)
            wait(kc, par)
            return accum(par, carry)

        fire(0, 0)

        def pair_body(p, carry):
            k0 = 2 * p
            carry = half(k0, 0, carry)
            carry = half(k0 + 1, 1, carry)
            return carry

        carry0 = ((zero,) * NACC, (zero,) * NACC)
        aa, bb = lax.fori_loop(0, nch // 2, pair_body, carry0)
        acc_a = (aa[0] + aa[1]) + (aa[2] + aa[3])
        acc_b = (bb[0] + bb[1]) + (bb[2] + bb[3])

        part_v[0] = acc_a
        part_v[1] = acc_b
        pltpu.sync_copy(part_v, shared.at[sid])
        plsc.subcore_barrier()

        @pl.when(sid == 0)
        def _():
            pltpu.sync_copy(shared, red_v)
            a = jnp.zeros((D,), jnp.float32)
            b = jnp.zeros((D,), jnp.float32)
            for i in range(NS):
                a = a + red_v[i, 0]
                b = b + red_v[i, 1]
            part_v[0] = a
            part_v[1] = b
            pltpu.sync_copy(part_v, out_hbm.at[cid])

    return k


def kernel(eb_input, eb_offset, W0, W1, W2, W3):
    del eb_offset
    n = eb_input.shape[0]
    idx2d = eb_input.astype(jnp.int32).reshape(n // GSZ, GSZ)
    parts = _make_kernel(n)(idx2d, W0, W1, W2, W3)
    return parts.sum(axis=0).reshape(2 * D)


# speedup vs baseline: 622.7814x; 8.0917x over previous
import functools

import jax
import jax.numpy as jnp
from jax import lax
from jax.experimental import pallas as pl
from jax.experimental.pallas import tpu as pltpu
from jax.experimental.pallas import tpu_sc as plsc

D = 16
NC = 2
NS = 16
NW = NC * NS
GSZ = 128
NBINS = 128 * 8192
BLK = 8192
ZCH = 8192


@functools.lru_cache(maxsize=None)
def _hist_kernel(n_idx):
    rows_per_w = n_idx // (NW * GSZ)
    bins_per_s = NBINS // NS

    mesh = plsc.VectorSubcoreMesh(core_axis_name="c", subcore_axis_name="s")

    @functools.partial(
        pl.kernel,
        out_type=jax.ShapeDtypeStruct((NC, NBINS), jnp.float32),
        mesh=mesh,
        compiler_params=pltpu.CompilerParams(use_tc_tiling_on_sc=False),
        scratch_types=[
            pltpu.VMEM((rows_per_w, GSZ), jnp.int32),
            pltpu.VMEM((GSZ,), jnp.float32),
            pltpu.VMEM((ZCH,), jnp.float32),
            pltpu.VMEM_SHARED((NBINS,), jnp.float32),
            pltpu.SemaphoreType.DMA,
        ],
    )
    def k(idx_hbm, out_hbm, idx_all, ones_v, zero_v, hist, sem):
        cid = lax.axis_index("c")
        sid = lax.axis_index("s")
        wid = sid * NC + cid
        one = jnp.full((D,), 1.0, jnp.float32)
        zero = jnp.zeros((D,), jnp.float32)

        idx_cp = pltpu.async_copy(
            idx_hbm.at[pl.ds(wid * rows_per_w, rows_per_w)], idx_all, sem)

        def fill_body(i, _):
            ones_v[pl.ds(i * D, D)] = one
            return 0

        lax.fori_loop(0, GSZ // D, fill_body, 0)

        def zfill_body(i, _):
            zero_v[pl.ds(i * D, D)] = zero
            return 0

        lax.fori_loop(0, ZCH // D, zfill_body, 0)

        def zcopy_body(i, _):
            pltpu.sync_copy(
                zero_v, hist.at[pl.ds(sid * bins_per_s + i * ZCH, ZCH)])
            return 0

        lax.fori_loop(0, bins_per_s // ZCH, zcopy_body, 0)
        plsc.subcore_barrier()
        idx_cp.wait()

        def scat_body(j, _):
            pltpu.sync_copy(ones_v, hist.at[idx_all.at[j]], add=True)
            return 0

        lax.fori_loop(0, rows_per_w, scat_body, 0)
        plsc.subcore_barrier()

        @pl.when(sid == 0)
        def _():
            pltpu.sync_copy(hist, out_hbm.at[cid])

    return k


def _wsum_body(cnt_ref, w0, w1, w2, w3, out_ref, acc_a, acc_b):
    i = pl.program_id(0)
    n_blk = pl.num_programs(0)
    c = jnp.sum(cnt_ref[...], axis=0, keepdims=True)
    col = i * BLK + lax.broadcasted_iota(jnp.int32, (1, BLK), 1)
    valid = col < 1000000
    pa = jnp.where(valid, (w0[...] + w2[...]) * c, 0.0)
    pb = jnp.where(valid, (w1[...] + w3[...]) * c, 0.0)

    @pl.when(i == 0)
    def _():
        acc_a[...] = pa
        acc_b[...] = pb

    @pl.when(i > 0)
    def _():
        acc_a[...] += pa
        acc_b[...] += pb

    @pl.when(i == n_blk - 1)
    def _():
        out_ref[0, :] = jnp.sum(acc_a[...], axis=1)
        out_ref[1, :] = jnp.sum(acc_b[...], axis=1)


@functools.lru_cache(maxsize=None)
def _wsum_kernel(n_rows):
    n_blk = (n_rows + BLK - 1) // BLK

    w_spec = pl.BlockSpec((D, BLK), lambda i: (0, i))
    return pl.pallas_call(
        _wsum_body,
        grid=(n_blk,),
        in_specs=[
            pl.BlockSpec((NC, BLK), lambda i: (0, i)),
            w_spec, w_spec, w_spec, w_spec,
        ],
        out_specs=pl.BlockSpec((NC, D), lambda i: (0, 0)),
        out_shape=jax.ShapeDtypeStruct((NC, D), jnp.float32),
        scratch_shapes=[
            pltpu.VMEM((D, BLK), jnp.float32),
            pltpu.VMEM((D, BLK), jnp.float32),
        ],
    )


def kernel(eb_input, eb_offset, W0, W1, W2, W3):
    del eb_offset
    n = eb_input.shape[0]
    idx2d = eb_input.astype(jnp.int32).reshape(n // GSZ, GSZ)
    counts = _hist_kernel(n)(idx2d)
    out2 = _wsum_kernel(W0.shape[0])(
        counts, W0.T, W1.T, W2.T, W3.T)
    return out2.reshape(NC * D)


# speedup vs baseline: 799.6968x; 1.2841x over previous
import functools

import jax
import jax.numpy as jnp
from jax import lax
from jax.experimental import pallas as pl
from jax.experimental.pallas import tpu as pltpu
from jax.experimental.pallas import tpu_sc as plsc

D = 16
NC = 2
NS = 16
NW = NC * NS
GSZ = 128
NBINS = 128 * 8192
BLK = 32768
ZCH = 8192


@functools.lru_cache(maxsize=None)
def _hist_kernel(n_idx):
    rows_per_w = n_idx // (NW * GSZ)
    bins_per_s = NBINS // NS

    mesh = plsc.VectorSubcoreMesh(core_axis_name="c", subcore_axis_name="s")

    @functools.partial(
        pl.kernel,
        out_type=jax.ShapeDtypeStruct((NC, NBINS), jnp.float32),
        mesh=mesh,
        compiler_params=pltpu.CompilerParams(use_tc_tiling_on_sc=False),
        scratch_types=[
            pltpu.VMEM((rows_per_w, GSZ), jnp.int32),
            pltpu.VMEM((GSZ,), jnp.float32),
            pltpu.VMEM((ZCH,), jnp.float32),
            pltpu.VMEM_SHARED((NBINS,), jnp.float32),
            pltpu.SemaphoreType.DMA,
        ],
    )
    def k(idx_hbm, out_hbm, idx_all, ones_v, zero_v, hist, sem):
        cid = lax.axis_index("c")
        sid = lax.axis_index("s")
        wid = sid * NC + cid
        one = jnp.full((D,), 1.0, jnp.float32)
        zero = jnp.zeros((D,), jnp.float32)

        idx_cp = pltpu.async_copy(
            idx_hbm.at[pl.ds(wid * rows_per_w, rows_per_w)], idx_all, sem)

        def fill_body(i, _):
            ones_v[pl.ds(i * D, D)] = one
            return 0

        lax.fori_loop(0, GSZ // D, fill_body, 0)

        def zfill_body(i, _):
            zero_v[pl.ds(i * D, D)] = zero
            return 0

        lax.fori_loop(0, ZCH // D, zfill_body, 0)

        def zcopy_body(i, _):
            pltpu.sync_copy(
                zero_v, hist.at[pl.ds(sid * bins_per_s + i * ZCH, ZCH)])
            return 0

        lax.fori_loop(0, bins_per_s // ZCH, zcopy_body, 0)
        plsc.subcore_barrier()
        idx_cp.wait()

        def scat_body(j, _):
            pltpu.sync_copy(ones_v, hist.at[idx_all.at[j]], add=True)
            return 0

        lax.fori_loop(0, rows_per_w, scat_body, 0)
        plsc.subcore_barrier()

        @pl.when(sid == 0)
        def _():
            pltpu.sync_copy(hist, out_hbm.at[cid])

    return k


@functools.lru_cache(maxsize=None)
def _wsum_kernel(n_rows):
    n_blk = (n_rows + BLK - 1) // BLK

    def body(cnt_ref, w0, w1, w2, w3, out_ref, acc_a, acc_b):
        i = pl.program_id(0)
        c = jnp.sum(cnt_ref[...], axis=0, keepdims=True)
        col = i * BLK + lax.broadcasted_iota(jnp.int32, (1, BLK), 1)
        valid = col < n_rows
        pa = jnp.where(valid, (w0[...] + w2[...]) * c, 0.0)
        pb = jnp.where(valid, (w1[...] + w3[...]) * c, 0.0)

        @pl.when(i == 0)
        def _():
            acc_a[...] = pa
            acc_b[...] = pb

        @pl.when(i > 0)
        def _():
            acc_a[...] += pa
            acc_b[...] += pb

        @pl.when(i == n_blk - 1)
        def _():
            out_ref[0, :] = jnp.sum(acc_a[...], axis=1)
            out_ref[1, :] = jnp.sum(acc_b[...], axis=1)

    w_spec = pl.BlockSpec((D, BLK), lambda i: (0, i))
    return pl.pallas_call(
        body,
        grid=(n_blk,),
        in_specs=[
            pl.BlockSpec((NC, BLK), lambda i: (0, i)),
            w_spec, w_spec, w_spec, w_spec,
        ],
        out_specs=pl.BlockSpec((NC, D), lambda i: (0, 0)),
        out_shape=jax.ShapeDtypeStruct((NC, D), jnp.float32),
        scratch_shapes=[
            pltpu.VMEM((D, BLK), jnp.float32),
            pltpu.VMEM((D, BLK), jnp.float32),
        ],
    )


def kernel(eb_input, eb_offset, W0, W1, W2, W3):
    del eb_offset
    n = eb_input.shape[0]
    idx2d = eb_input.astype(jnp.int32).reshape(n // GSZ, GSZ)
    counts = _hist_kernel(n)(idx2d)
    out2 = _wsum_kernel(W0.shape[0])(
        counts, W0.T, W1.T, W2.T, W3.T)
    return out2.reshape(NC * D)


# speedup vs baseline: 810.1664x; 1.0131x over previous
import functools

import jax
import jax.numpy as jnp
from jax import lax
from jax.experimental import pallas as pl
from jax.experimental.pallas import tpu as pltpu
from jax.experimental.pallas import tpu_sc as plsc

D = 16
NC = 2
NS = 16
NW = NC * NS
GSZ = 128
NBINS = 128 * 8192
BLK = 65536
ZCH = 8192


@functools.lru_cache(maxsize=None)
def _hist_kernel(n_idx):
    rows_per_w = n_idx // (NW * GSZ)
    bins_per_s = NBINS // NS

    mesh = plsc.VectorSubcoreMesh(core_axis_name="c", subcore_axis_name="s")

    @functools.partial(
        pl.kernel,
        out_type=jax.ShapeDtypeStruct((NC, NBINS), jnp.float32),
        mesh=mesh,
        compiler_params=pltpu.CompilerParams(use_tc_tiling_on_sc=False),
        scratch_types=[
            pltpu.VMEM((rows_per_w, GSZ), jnp.int32),
            pltpu.VMEM((GSZ,), jnp.float32),
            pltpu.VMEM((ZCH,), jnp.float32),
            pltpu.VMEM_SHARED((NBINS,), jnp.float32),
            pltpu.SemaphoreType.DMA,
        ],
    )
    def k(idx_hbm, out_hbm, idx_all, ones_v, zero_v, hist, sem):
        cid = lax.axis_index("c")
        sid = lax.axis_index("s")
        wid = sid * NC + cid
        one = jnp.full((D,), 1.0, jnp.float32)
        zero = jnp.zeros((D,), jnp.float32)

        idx_cp = pltpu.async_copy(
            idx_hbm.at[pl.ds(wid * rows_per_w, rows_per_w)], idx_all, sem)

        def fill_body(i, _):
            ones_v[pl.ds(i * D, D)] = one
            return 0

        lax.fori_loop(0, GSZ // D, fill_body, 0)

        def zfill_body(i, _):
            zero_v[pl.ds(i * D, D)] = zero
            return 0

        lax.fori_loop(0, ZCH // D, zfill_body, 0)

        def zcopy_body(i, _):
            pltpu.sync_copy(
                zero_v, hist.at[pl.ds(sid * bins_per_s + i * ZCH, ZCH)])
            return 0

        lax.fori_loop(0, bins_per_s // ZCH, zcopy_body, 0)
        plsc.subcore_barrier()
        idx_cp.wait()

        def scat_body(j, _):
            pltpu.sync_copy(ones_v, hist.at[idx_all.at[j]], add=True)
            return 0

        lax.fori_loop(0, rows_per_w, scat_body, 0)
        plsc.subcore_barrier()

        @pl.when(sid == 0)
        def _():
            pltpu.sync_copy(hist, out_hbm.at[cid])

    return k


@functools.lru_cache(maxsize=None)
def _wsum_kernel(n_rows):
    n_blk = (n_rows + BLK - 1) // BLK

    def body(cnt_ref, w0, w1, w2, w3, out_ref, acc_a, acc_b):
        i = pl.program_id(0)
        c = jnp.sum(cnt_ref[...], axis=0, keepdims=True)
        col = i * BLK + lax.broadcasted_iota(jnp.int32, (1, BLK), 1)
        valid = col < n_rows
        pa = jnp.where(valid, (w0[...] + w2[...]) * c, 0.0)
        pb = jnp.where(valid, (w1[...] + w3[...]) * c, 0.0)

        @pl.when(i == 0)
        def _():
            acc_a[...] = pa
            acc_b[...] = pb

        @pl.when(i > 0)
        def _():
            acc_a[...] += pa
            acc_b[...] += pb

        @pl.when(i == n_blk - 1)
        def _():
            out_ref[0, :] = jnp.sum(acc_a[...], axis=1)
            out_ref[1, :] = jnp.sum(acc_b[...], axis=1)

    w_spec = pl.BlockSpec((D, BLK), lambda i: (0, i))
    return pl.pallas_call(
        body,
        grid=(n_blk,),
        in_specs=[
            pl.BlockSpec((NC, BLK), lambda i: (0, i)),
            w_spec, w_spec, w_spec, w_spec,
        ],
        out_specs=pl.BlockSpec((NC, D), lambda i: (0, 0)),
        out_shape=jax.ShapeDtypeStruct((NC, D), jnp.float32),
        scratch_shapes=[
            pltpu.VMEM((D, BLK), jnp.float32),
            pltpu.VMEM((D, BLK), jnp.float32),
        ],
    )


def kernel(eb_input, eb_offset, W0, W1, W2, W3):
    del eb_offset
    n = eb_input.shape[0]
    idx2d = eb_input.astype(jnp.int32).reshape(n // GSZ, GSZ)
    counts = _hist_kernel(n)(idx2d)
    out2 = _wsum_kernel(W0.shape[0])(
        counts, W0.T, W1.T, W2.T, W3.T)
    return out2.reshape(NC * D)


# speedup vs baseline: 874.3889x; 1.0793x over previous
import functools

import jax
import jax.numpy as jnp
from jax import lax
from jax.experimental import pallas as pl
from jax.experimental.pallas import tpu as pltpu
from jax.experimental.pallas import tpu_sc as plsc

D = 16
NC = 2
NS = 16
NW = NC * NS
GSZ = 128
NBINS = 128 * 8192
BLK = 65536
ZCH = 8192


@functools.lru_cache(maxsize=None)
def _hist_kernel(n_idx):
    rows_per_w = n_idx // (NW * GSZ)
    bins_per_s = NBINS // NS

    mesh = plsc.VectorSubcoreMesh(core_axis_name="c", subcore_axis_name="s")

    @functools.partial(
        pl.kernel,
        out_type=jax.ShapeDtypeStruct((NC, NBINS), jnp.float32),
        mesh=mesh,
        compiler_params=pltpu.CompilerParams(use_tc_tiling_on_sc=False),
        scratch_types=[
            pltpu.VMEM((rows_per_w, GSZ), jnp.int32),
            pltpu.VMEM((GSZ,), jnp.float32),
            pltpu.VMEM_SHARED((NBINS,), jnp.float32),
            pltpu.SemaphoreType.DMA,
            pltpu.SemaphoreType.DMA,
            pltpu.SemaphoreType.DMA,
        ],
    )
    def k(idx_hbm, zeros_hbm, out_hbm, idx_all, ones_v, hist,
          semi, semz, sems):
        cid = lax.axis_index("c")
        sid = lax.axis_index("s")
        wid = sid * NC + cid
        one = jnp.full((D,), 1.0, jnp.float32)

        idx_cp = pltpu.async_copy(
            idx_hbm.at[pl.ds(wid * rows_per_w, rows_per_w)], idx_all, semi)
        zsl = pl.ds(sid * bins_per_s, bins_per_s)
        z_cp = pltpu.async_copy(zeros_hbm.at[zsl], hist.at[zsl], semz)

        def fill_body(i, _):
            ones_v[pl.ds(i * D, D)] = one
            return 0

        lax.fori_loop(0, GSZ // D, fill_body, 0)
        z_cp.wait()
        plsc.subcore_barrier()
        idx_cp.wait()

        ring = 8

        def scat_body(j, _):
            pltpu.async_copy(ones_v, hist.at[idx_all.at[j]], sems, add=True)

            @pl.when(j >= ring)
            def _():
                pltpu.make_async_copy(
                    ones_v, hist.at[idx_all.at[j - ring]], sems).wait()

            return 0

        lax.fori_loop(0, rows_per_w, scat_body, 0)
        for t in range(ring):
            pltpu.make_async_copy(
                ones_v, hist.at[idx_all.at[rows_per_w - ring + t]],
                sems).wait()
        plsc.subcore_barrier()

        @pl.when(sid == 0)
        def _():
            pltpu.sync_copy(hist, out_hbm.at[cid])

    return k


@functools.lru_cache(maxsize=None)
def _wsum_kernel(n_rows):
    n_blk = (n_rows + BLK - 1) // BLK

    def body(cnt_ref, w0, w1, w2, w3, out_ref, acc_a, acc_b):
        i = pl.program_id(0)
        c = jnp.sum(cnt_ref[...], axis=0, keepdims=True)
        col = i * BLK + lax.broadcasted_iota(jnp.int32, (1, BLK), 1)
        valid = col < n_rows
        pa = jnp.where(valid, (w0[...] + w2[...]) * c, 0.0)
        pb = jnp.where(valid, (w1[...] + w3[...]) * c, 0.0)

        @pl.when(i == 0)
        def _():
            acc_a[...] = pa
            acc_b[...] = pb

        @pl.when(i > 0)
        def _():
            acc_a[...] += pa
            acc_b[...] += pb

        @pl.when(i == n_blk - 1)
        def _():
            out_ref[0, :] = jnp.sum(acc_a[...], axis=1)
            out_ref[1, :] = jnp.sum(acc_b[...], axis=1)

    w_spec = pl.BlockSpec((D, BLK), lambda i: (0, i))
    return pl.pallas_call(
        body,
        grid=(n_blk,),
        in_specs=[
            pl.BlockSpec((NC, BLK), lambda i: (0, i)),
            w_spec, w_spec, w_spec, w_spec,
        ],
        out_specs=pl.BlockSpec((NC, D), lambda i: (0, 0)),
        out_shape=jax.ShapeDtypeStruct((NC, D), jnp.float32),
        scratch_shapes=[
            pltpu.VMEM((D, BLK), jnp.float32),
            pltpu.VMEM((D, BLK), jnp.float32),
        ],
    )


def kernel(eb_input, eb_offset, W0, W1, W2, W3):
    del eb_offset
    n = eb_input.shape[0]
    idx2d = eb_input.astype(jnp.int32).reshape(n // GSZ, GSZ)
    zeros = jnp.zeros((NBINS,), jnp.float32)
    counts = _hist_kernel(n)(idx2d, zeros)
    out2 = _wsum_kernel(W0.shape[0])(
        counts, W0.T, W1.T, W2.T, W3.T)
    return out2.reshape(NC * D)


# speedup vs baseline: 891.9728x; 1.0201x over previous
import functools

import jax
import jax.numpy as jnp
from jax import lax
from jax.experimental import pallas as pl
from jax.experimental.pallas import tpu as pltpu
from jax.experimental.pallas import tpu_sc as plsc

D = 16
NC = 2
NS = 16
NW = NC * NS
GSZ = 128
NBINS = 128 * 8192
BLK = 16384
CUT = 589824
SC_END = 999424
QW = 12800
QC = 256


@functools.lru_cache(maxsize=None)
def _hist_kernel(n_idx):
    rows_per_w = n_idx // (NW * GSZ)
    bins_per_s = NBINS // NS

    mesh = plsc.VectorSubcoreMesh(core_axis_name="c", subcore_axis_name="s")

    @functools.partial(
        pl.kernel,
        out_type=jax.ShapeDtypeStruct((NC * NBINS,), jnp.float32),
        mesh=mesh,
        compiler_params=pltpu.CompilerParams(use_tc_tiling_on_sc=False),
        scratch_types=[
            pltpu.VMEM((rows_per_w, GSZ), jnp.int32),
            pltpu.VMEM((GSZ,), jnp.float32),
            pltpu.VMEM_SHARED((NBINS,), jnp.float32),
            pltpu.SemaphoreType.DMA,
            pltpu.SemaphoreType.DMA,
            pltpu.SemaphoreType.DMA,
        ],
    )
    def k(idx_hbm, zeros_hbm, out_hbm, idx_all, ones_v, hist,
          semi, semz, sems):
        cid = lax.axis_index("c")
        sid = lax.axis_index("s")
        wid = sid * NC + cid
        one = jnp.full((D,), 1.0, jnp.float32)

        idx_cp = pltpu.async_copy(
            idx_hbm.at[pl.ds(wid * rows_per_w, rows_per_w)], idx_all, semi)
        zsl = pl.ds(sid * bins_per_s, bins_per_s)
        z_cp = pltpu.async_copy(zeros_hbm.at[zsl], hist.at[zsl], semz)

        def fill_body(i, _):
            ones_v[pl.ds(i * D, D)] = one
            return 0

        lax.fori_loop(0, GSZ // D, fill_body, 0)
        z_cp.wait()
        plsc.subcore_barrier()
        idx_cp.wait()

        ring = 8

        def scat_body(j, _):
            pltpu.async_copy(ones_v, hist.at[idx_all.at[j]], sems, add=True)

            @pl.when(j >= ring)
            def _():
                pltpu.make_async_copy(
                    ones_v, hist.at[idx_all.at[j - ring]], sems).wait()

            return 0

        lax.fori_loop(0, rows_per_w, scat_body, 0)
        for t in range(ring):
            pltpu.make_async_copy(
                ones_v, hist.at[idx_all.at[rows_per_w - ring + t]],
                sems).wait()
        plsc.subcore_barrier()

        @pl.when(sid == 0)
        def _():
            pltpu.sync_copy(hist, out_hbm.at[pl.ds(cid * NBINS, NBINS)])

    return k


@functools.lru_cache(maxsize=None)
def _tc_wsum_kernel(n_rows):
    n_blk = CUT // BLK + 1
    tail_idx = SC_END // BLK

    def body(cnt_ref, cnt_t, w0, w1, w2, w3, w0t, w1t, w2t, w3t,
             out_ref, acc_a, acc_b):
        i = pl.program_id(0)
        c = jnp.sum(cnt_ref[...], axis=0, keepdims=True)
        col = i * BLK + lax.broadcasted_iota(jnp.int32, (1, BLK), 1)
        valid = col < CUT
        pa = jnp.where(valid, (w0[...] + w2[...]) * c, 0.0)
        pb = jnp.where(valid, (w1[...] + w3[...]) * c, 0.0)

        @pl.when(i == 0)
        def _():
            acc_a[...] = pa
            acc_b[...] = pb

        @pl.when(i > 0)
        def _():
            acc_a[...] += pa
            acc_b[...] += pb

        @pl.when(i == n_blk - 1)
        def _():
            ct = jnp.sum(cnt_t[...], axis=0, keepdims=True)
            colt = (tail_idx * BLK
                    + lax.broadcasted_iota(jnp.int32, (1, BLK), 1))
            vt = (colt >= SC_END) & (colt < n_rows)
            acc_a[...] += jnp.where(vt, (w0t[...] + w2t[...]) * ct, 0.0)
            acc_b[...] += jnp.where(vt, (w1t[...] + w3t[...]) * ct, 0.0)
            out_ref[0, :] = jnp.sum(acc_a[...], axis=1)
            out_ref[1, :] = jnp.sum(acc_b[...], axis=1)

    w_spec = pl.BlockSpec((D, BLK), lambda i: (0, i))
    w_tail = pl.BlockSpec((D, BLK), lambda i: (0, tail_idx))
    return pl.pallas_call(
        body,
        grid=(n_blk,),
        in_specs=[
            pl.BlockSpec((NC, BLK), lambda i: (0, i)),
            pl.BlockSpec((NC, BLK), lambda i: (0, tail_idx)),
            w_spec, w_spec, w_spec, w_spec,
            w_tail, w_tail, w_tail, w_tail,
        ],
        out_specs=pl.BlockSpec((NC, D), lambda i: (0, 0)),
        out_shape=jax.ShapeDtypeStruct((NC, D), jnp.float32),
        scratch_shapes=[
            pltpu.VMEM((D, BLK), jnp.float32),
            pltpu.VMEM((D, BLK), jnp.float32),
        ],
    )


@functools.lru_cache(maxsize=None)
def _sc_wsum_kernel():
    mesh = plsc.VectorSubcoreMesh(core_axis_name="c", subcore_axis_name="s")
    nt = QC // GSZ
    wbuf = pltpu.VMEM((nt, 8, GSZ), jnp.float32)
    cbuf = pltpu.VMEM((QC,), jnp.float32)

    @functools.partial(
        pl.kernel,
        out_type=jax.ShapeDtypeStruct((NC, 2 * D, D), jnp.float32),
        mesh=mesh,
        compiler_params=pltpu.CompilerParams(use_tc_tiling_on_sc=True),
        scratch_types=(
            [wbuf] * 16
            + [cbuf] * 4
            + [
                pltpu.VMEM((2 * D, D), jnp.float32),
                pltpu.VMEM((NS, 2 * D, D), jnp.float32),
                pltpu.VMEM_SHARED((NS, 2 * D, D), jnp.float32),
                pltpu.SemaphoreType.DMA,
                pltpu.SemaphoreType.DMA,
            ]
        ),
    )
    def k(cnt_hbm, w0, w1, w2, w3, out_hbm, *refs):
        wbufs = [[[refs[p * 8 + t * 2 + s] for s in range(2)]
                  for t in range(4)] for p in range(2)]
        cbufs = [[refs[16 + p * 2 + h] for h in range(2)] for p in range(2)]
        accv, red_v, shared, sem0, sem1 = refs[20:]
        sems = (sem0, sem1)
        ws = (w0, w1, w2, w3)

        cid = lax.axis_index("c")
        sid = lax.axis_index("s")
        wid = sid * NC + cid
        base = CUT + wid * QW
        nch = QW // QC
        zero = jnp.zeros((D,), jnp.float32)

        def fire(kc, par):
            col0 = base + kc * QC
            for t in range(4):
                for s in range(2):
                    for tt in range(nt):
                        pltpu.async_copy(
                            ws[t].at[pl.ds(s * 8, 8),
                                     pl.ds(col0 + tt * GSZ, GSZ)],
                            wbufs[par][t][s].at[tt], sems[par])
            for h in range(2):
                pltpu.async_copy(
                    cnt_hbm.at[pl.ds(h * NBINS + col0, QC)],
                    cbufs[par][h], sems[par])

        def wait(kc, par):
            col0 = base + kc * QC
            for t in range(4):
                for s in range(2):
                    for tt in range(nt):
                        pltpu.make_async_copy(
                            ws[t].at[pl.ds(s * 8, 8),
                                     pl.ds(col0 + tt * GSZ, GSZ)],
                            wbufs[par][t][s].at[tt], sems[par]).wait()
            for h in range(2):
                pltpu.make_async_copy(
                    cnt_hbm.at[pl.ds(h * NBINS + col0, QC)],
                    cbufs[par][h], sems[par]).wait()

        def accum(par, carry):
            def m_body(m, carry2):
                acc = list(carry2)
                tt = m // (GSZ // D)
                off = (m % (GSZ // D)) * D
                cv = (cbufs[par][0][pl.ds(m * D, D)]
                      + cbufs[par][1][pl.ds(m * D, D)])
                for t in range(4):
                    g = t % 2
                    for s in range(2):
                        for r in range(8):
                            a = g * D + s * 8 + r
                            acc[a] = (acc[a]
                                      + wbufs[par][t][s][tt, r, pl.ds(off, D)]
                                      * cv)
                return tuple(acc)

            return lax.fori_loop(0, QC // D, m_body, carry)

        def half(kc, par, carry):
            @pl.when(kc + 1 < nch)
            def _():
                fire(kc + 1, 1 - par)
            wait(kc, par)
            return accum(par, carry)

        fire(0, 0)

        def pair_body(p, carry):
            carry = half(2 * p, 0, carry)
            carry = half(2 * p + 1, 1, carry)
            return carry

        carry = lax.fori_loop(0, nch // 2, pair_body, (zero,) * (2 * D))
        for a in range(2 * D):
            accv[a] = carry[a]
        pltpu.sync_copy(accv, shared.at[sid])
        plsc.subcore_barrier()

        @pl.when(sid == 0)
        def _():
            pltpu.sync_copy(shared, red_v)
            for a in range(2 * D):
                v = jnp.zeros((D,), jnp.float32)
                for i in range(NS):
                    v = v + red_v[i, a]
                accv[a] = v
            pltpu.sync_copy(accv, out_hbm.at[cid])

    return k


def kernel(eb_input, eb_offset, W0, W1, W2, W3):
    del eb_offset
    n = eb_input.shape[0]
    idx2d = eb_input.astype(jnp.int32).reshape(n // GSZ, GSZ)
    zeros = jnp.zeros((NBINS,), jnp.float32)
    cnt_flat = _hist_kernel(n)(idx2d, zeros)
    wt = (W0.T, W1.T, W2.T, W3.T)
    cnt2 = cnt_flat.reshape(NC, NBINS)
    tc_out = _tc_wsum_kernel(W0.shape[0])(
        cnt2, cnt2, *wt, *wt)
    sc_out = _sc_wsum_kernel()(cnt_flat, *wt)
    return tc_out.reshape(NC * D) + sc_out.sum(axis=(0, 2))
